# Initial kernel scaffold; baseline (speedup 1.0000x reference)
import jax
import jax.numpy as jnp
from jax import lax
from jax.experimental import pallas as pl
from jax.experimental.pallas import tpu as pltpu
from jax.experimental.pallas import tpu_sc as plsc

N_ATOMS_K = 1_600_000
N_SEG = 100_000
N_SPECIES_PAD = 128

NC = 2
NS = 16
L = 16
NW = NC * NS
APT = N_ATOMS_K // NW
BLK = 2_000
NBLK = APT // BLK
ACC_PAD = 102_400
SLC = ACC_PAD // NS


def _sc_body(le_hbm, z_hbm, seg_hbm, s1_hbm, s2_hbm, out_hbm,
             scale_v, s2_v, le_v, z_v, seg_v, e_v, zero_v, acc_sh):
    cid = lax.axis_index("c")
    sid = lax.axis_index("s")
    wid = cid * NS + sid

    pltpu.sync_copy(s1_hbm, scale_v)
    pltpu.sync_copy(s2_hbm, s2_v)

    @pl.loop(0, N_SPECIES_PAD, step=L)
    def _(i):
        scale_v[pl.ds(i, L)] = scale_v[pl.ds(i, L)] * s2_v[pl.ds(i, L)]

    @pl.loop(0, SLC, step=L)
    def _(i):
        zero_v[pl.ds(i, L)] = jnp.zeros((L,), jnp.float32)

    pltpu.sync_copy(zero_v, acc_sh.at[pl.ds(sid * SLC, SLC)])
    plsc.subcore_barrier()

    base0 = wid * APT

    @pl.loop(0, NBLK)
    def _(b):
        base = base0 + b * BLK
        pltpu.sync_copy(le_hbm.at[pl.ds(base, BLK)], le_v)
        pltpu.sync_copy(z_hbm.at[pl.ds(base, BLK)], z_v)
        pltpu.sync_copy(seg_hbm.at[pl.ds(base, BLK)], seg_v)

        @pl.loop(0, BLK, step=L)
        def _(i):
            z = z_v[pl.ds(i, L)]
            sv = plsc.load_gather(scale_v, [z])
            e_v[pl.ds(i, L)] = le_v[pl.ds(i, L)] * sv

        pltpu.sync_copy(e_v, acc_sh.at[seg_v], add=True)

    plsc.subcore_barrier()
    pltpu.sync_copy(acc_sh.at[pl.ds(sid * SLC, SLC)],
                    out_hbm.at[cid, pl.ds(sid * SLC, SLC)])


def _combine_body(p_ref, o_ref):
    o_ref[...] = p_ref[0] + p_ref[1]


@jax.jit
def _impl(local_energies, Z, segment_ids, s1p, s2p):
    mesh = plsc.VectorSubcoreMesh(core_axis_name="c", subcore_axis_name="s")
    sc_call = pl.kernel(
        _sc_body,
        out_type=jax.ShapeDtypeStruct((NC, ACC_PAD), jnp.float32),
        mesh=mesh,
        scratch_types=[
            pltpu.VMEM((N_SPECIES_PAD,), jnp.float32),
            pltpu.VMEM((N_SPECIES_PAD,), jnp.float32),
            pltpu.VMEM((BLK,), jnp.float32),
            pltpu.VMEM((BLK,), jnp.int32),
            pltpu.VMEM((BLK,), jnp.int32),
            pltpu.VMEM((BLK,), jnp.float32),
            pltpu.VMEM((SLC,), jnp.float32),
            pltpu.VMEM_SHARED((ACC_PAD,), jnp.float32),
        ],
    )
    partial = sc_call(local_energies, Z, segment_ids, s1p, s2p)
    p3 = partial.reshape(NC, ACC_PAD // 128, 128)
    tot = pl.pallas_call(
        _combine_body,
        out_shape=jax.ShapeDtypeStruct((ACC_PAD // 128, 128), jnp.float32),
    )(p3)
    return tot.reshape(-1)[:N_SEG]


def kernel(local_energies, Z, segment_ids, scale1, scale2):
    s1p = jnp.pad(scale1, (0, N_SPECIES_PAD - scale1.shape[0]))
    s2p = jnp.pad(scale2, (0, N_SPECIES_PAD - scale2.shape[0]))
    return _impl(local_energies, Z, segment_ids, s1p, s2p)


# speedup vs baseline: 215.8807x; 215.8807x over previous
import dataclasses

import jax
import jax.numpy as jnp
from jax import lax
from jax.experimental import pallas as pl
from jax.experimental.pallas import tpu as pltpu
from jax.experimental.pallas import tpu_sc as plsc

N_ATOMS_K = 1_600_000
N_SEG = 100_000
N_SPECIES_PAD = 128

NC = 2
NS = 16
L = 16
NW = NC * NS
APT = N_ATOMS_K // NW
BLK = 2_000
NBLK = APT // BLK
ACC_PAD = 102_400
SLC = ACC_PAD // NS


def _sc_body(le_hbm, z_hbm, seg_hbm, s1_hbm, s2_hbm, out_hbm,
             scale_v, s2_v, le_v, z_v, seg_v, e_v, zero_v, acc_sh):
    cid = lax.axis_index("c")
    sid = lax.axis_index("s")
    wid = cid * NS + sid

    pltpu.sync_copy(s1_hbm, scale_v)
    pltpu.sync_copy(s2_hbm, s2_v)

    @pl.loop(0, N_SPECIES_PAD, step=L)
    def _(i):
        scale_v[pl.ds(i, L)] = scale_v[pl.ds(i, L)] * s2_v[pl.ds(i, L)]

    @pl.loop(0, SLC, step=L)
    def _(i):
        zero_v[pl.ds(i, L)] = jnp.zeros((L,), jnp.float32)

    pltpu.sync_copy(zero_v, acc_sh.at[pl.ds(sid * SLC, SLC)])
    plsc.subcore_barrier()

    base0 = wid * APT

    @pl.loop(0, NBLK)
    def _(b):
        base = base0 + b * BLK
        pltpu.sync_copy(le_hbm.at[pl.ds(base, BLK)], le_v)
        pltpu.sync_copy(z_hbm.at[pl.ds(base, BLK)], z_v)
        pltpu.sync_copy(seg_hbm.at[pl.ds(base, BLK)], seg_v)

        @pl.loop(0, BLK, step=L)
        def _(i):
            z = z_v[pl.ds(i, L)]
            sv = plsc.load_gather(scale_v, [z])
            e_v[pl.ds(i, L)] = le_v[pl.ds(i, L)] * sv

        pltpu.sync_copy(e_v, acc_sh.at[seg_v], add=True)

    plsc.subcore_barrier()
    pltpu.sync_copy(acc_sh.at[pl.ds(sid * SLC, SLC)],
                    out_hbm.at[cid, pl.ds(sid * SLC, SLC)])


def _combine_body(p_ref, o_ref):
    o_ref[...] = p_ref[0] + p_ref[1]


@jax.jit
def _impl(local_energies, Z, segment_ids, s1p, s2p):
    mesh = plsc.VectorSubcoreMesh(core_axis_name="c", subcore_axis_name="s")
    cp = pltpu.CompilerParams()
    if "needs_layout_passes" in pltpu.CompilerParams.__dataclass_fields__:
        cp = dataclasses.replace(cp, needs_layout_passes=False)
    sc_call = pl.kernel(
        _sc_body,
        out_type=jax.ShapeDtypeStruct((NC, ACC_PAD), jnp.float32),
        mesh=mesh,
        scratch_types=[
            pltpu.VMEM((N_SPECIES_PAD,), jnp.float32),
            pltpu.VMEM((N_SPECIES_PAD,), jnp.float32),
            pltpu.VMEM((BLK,), jnp.float32),
            pltpu.VMEM((BLK,), jnp.int32),
            pltpu.VMEM((BLK,), jnp.int32),
            pltpu.VMEM((BLK,), jnp.float32),
            pltpu.VMEM((SLC,), jnp.float32),
            pltpu.VMEM_SHARED((ACC_PAD,), jnp.float32),
        ],
        compiler_params=cp,
    )
    partial = sc_call(local_energies, Z, segment_ids, s1p, s2p)
    p3 = partial.reshape(NC, ACC_PAD // 128, 128)
    tot = pl.pallas_call(
        _combine_body,
        out_shape=jax.ShapeDtypeStruct((ACC_PAD // 128, 128), jnp.float32),
    )(p3)
    return tot.reshape(-1)[:N_SEG]


def kernel(local_energies, Z, segment_ids, scale1, scale2):
    s1p = jnp.pad(scale1, (0, N_SPECIES_PAD - scale1.shape[0]))
    s2p = jnp.pad(scale2, (0, N_SPECIES_PAD - scale2.shape[0]))
    return _impl(local_energies, Z, segment_ids, s1p, s2p)
